# baseline (speedup 1.0000x reference)
import jax
import jax.numpy as jnp
from jax import lax
from jax.experimental import pallas as pl
from jax.experimental.pallas import tpu as pltpu
from jax.experimental.pallas import tpu_sc as plsc

D = 32
QW = 128
C = 128
J = 2
SJ = C // J
G16 = C // 16
NIB = 4
NC, NS = 2, 16
NW = NC * NS


def _gather_body(idx_hbm, idxq_hbm, tbl_hbm, out_hbm, idx_v, qid_v, rowsA, rowsB,
                 isem, qsem, gsem, osem):
    wid = lax.axis_index("s") * NC + lax.axis_index("c")
    n_per_w = idx_hbm.shape[0] // NW
    base0 = wid * n_per_w
    nch = n_per_w // C
    iota16 = lax.iota(jnp.int32, 16)

    def fire_idx(g, ib):
        pltpu.async_copy(
            idx_hbm.at[pl.ds(pl.multiple_of(base0 + g * C, C), C)],
            idx_v.at[ib], isem.at[ib],
        )
        pltpu.async_copy(
            idxq_hbm.at[pl.ds(pl.multiple_of(base0 + g * C, C), C)],
            qid_v.at[ib], qsem.at[ib],
        )

    def wait_idx(ib):
        pltpu.make_async_copy(
            idx_hbm.at[pl.ds(pl.multiple_of(base0, C), C)],
            idx_v.at[ib], isem.at[ib],
        ).wait()
        pltpu.make_async_copy(
            idxq_hbm.at[pl.ds(pl.multiple_of(base0, C), C)],
            qid_v.at[ib], qsem.at[ib],
        ).wait()

    def fire_gathers(ib, ra):
        for j in range(J):
            pltpu.async_copy(
                tbl_hbm.at[qid_v.at[ib, pl.ds(j * SJ, SJ)]],
                rowsA.at[ra, pl.ds(j * SJ, SJ)],
                gsem.at[ra],
            )

    def drain_gathers(ra):
        for j in range(J):
            pltpu.make_async_copy(
                tbl_hbm.at[qid_v.at[0, pl.ds(0, SJ)]],
                rowsA.at[ra, pl.ds(j * SJ, SJ)],
                gsem.at[ra],
            ).wait()

    def extract(ib, ra, rb):
        def ebody(m, carry):
            iv = idx_v[ib, pl.ds(m * 16, 16)]
            colv0 = lax.shift_left(jnp.bitwise_and(iv, 3), 5)
            rows16 = m * 16 + iota16
            for c in range(D):
                x = plsc.load_gather(rowsA.at[ra], [rows16, colv0 + c])
                plsc.store_scatter(
                    rowsB.at[rb], [rows16, jnp.full((16,), c, jnp.int32)], x
                )
            return carry
        lax.fori_loop(0, G16, ebody, 0)

    def fire_store(g, rb):
        pltpu.async_copy(
            rowsB.at[rb],
            out_hbm.at[pl.ds(pl.multiple_of(base0 + g * C, C), C)],
            osem.at[rb],
        )

    def wait_store(rb):
        pltpu.make_async_copy(
            rowsB.at[rb],
            out_hbm.at[pl.ds(pl.multiple_of(base0, C), C)],
            osem.at[rb],
        ).wait()

    for g in range(2):
        fire_idx(g, g)

    def outer(i, carry):
        for u in range(4):
            g = 4 * i + u
            ib = u
            ra = u % 2
            wait_idx(ib)
            fire_gathers(ib, ra)

            @pl.when(g + 2 < nch)
            def _():
                fire_idx(g + 2, (u + 2) % 4)

            @pl.when(g >= 1)
            def _():
                drain_gathers(1 - ra)

                @pl.when(g >= 3)
                def _():
                    wait_store(1 - ra)

                extract((u + 3) % 4, 1 - ra, 1 - ra)
                fire_store(g - 1, 1 - ra)
        return carry

    lax.fori_loop(0, nch // 4, outer, 0)
    last = (nch - 1) % 2
    drain_gathers(last)
    wait_store(last)
    extract((nch - 1) % 4, last, last)
    fire_store(nch - 1, last)
    for rb in range(2):
        wait_store(rb)


def kernel(indexes, index2vec_weight):
    n = indexes.shape[0]
    assert n % (NW * C * 4) == 0
    tblq = index2vec_weight.reshape(-1, QW)
    mesh = plsc.VectorSubcoreMesh(core_axis_name="c", subcore_axis_name="s")
    f = pl.kernel(
        _gather_body,
        out_type=jax.ShapeDtypeStruct((n, D), jnp.float32),
        mesh=mesh,
        scratch_types=[
            pltpu.VMEM((NIB, C), jnp.int32),
            pltpu.VMEM((NIB, C), jnp.int32),
            pltpu.VMEM((2, C, QW), jnp.float32),
            pltpu.VMEM((2, C, D), jnp.float32),
            pltpu.SemaphoreType.DMA((NIB,)),
            pltpu.SemaphoreType.DMA((NIB,)),
            pltpu.SemaphoreType.DMA((2,)),
            pltpu.SemaphoreType.DMA((2,)),
        ],
        compiler_params=pltpu.CompilerParams(use_tc_tiling_on_sc=True, needs_layout_passes=False),
    )
    return f(indexes, indexes >> 2, tblq)


# speedup vs baseline: 1.9602x; 1.9602x over previous
import jax
import jax.numpy as jnp
from jax import lax
from jax.experimental import pallas as pl
from jax.experimental.pallas import tpu as pltpu
from jax.experimental.pallas import tpu_sc as plsc

D = 32
QW = 128
C = 128
J = 2
SJ = C // J
G16 = C // 16
NIB = 4
NC, NS = 2, 16
NW = NC * NS


def _gather_body(idx_hbm, idxq_hbm, tbl_hbm, out_hbm, idx_v, qid_v, rowsA, rowsB,
                 isem, qsem, gsem, osem):
    wid = lax.axis_index("s") * NC + lax.axis_index("c")
    n_per_w = idx_hbm.shape[0] // NW
    base0 = wid * n_per_w
    nch = n_per_w // C
    iota16 = lax.iota(jnp.int32, 16)

    def fire_idx(g, ib):
        pltpu.async_copy(
            idx_hbm.at[pl.ds(pl.multiple_of(base0 + g * C, C), C)],
            idx_v.at[ib], isem.at[ib],
        )
        pltpu.async_copy(
            idxq_hbm.at[pl.ds(pl.multiple_of(base0 + g * C, C), C)],
            qid_v.at[ib], qsem.at[ib],
        )

    def wait_idx(ib):
        pltpu.make_async_copy(
            idx_hbm.at[pl.ds(pl.multiple_of(base0, C), C)],
            idx_v.at[ib], isem.at[ib],
        ).wait()
        pltpu.make_async_copy(
            idxq_hbm.at[pl.ds(pl.multiple_of(base0, C), C)],
            qid_v.at[ib], qsem.at[ib],
        ).wait()

    def fire_gathers(ib, ra):
        for j in range(J):
            pltpu.async_copy(
                tbl_hbm.at[qid_v.at[ib, pl.ds(j * SJ, SJ)]],
                rowsA.at[ra, pl.ds(j * SJ, SJ)],
                gsem.at[ra],
            )

    def drain_gathers(ra):
        for j in range(J):
            pltpu.make_async_copy(
                tbl_hbm.at[qid_v.at[0, pl.ds(0, SJ)]],
                rowsA.at[ra, pl.ds(j * SJ, SJ)],
                gsem.at[ra],
            ).wait()

    def extract(ib, ra, rb):
        def ebody(m, carry):
            iv = idx_v[ib, pl.ds(m * 16, 16)]
            qv = lax.shift_left(jnp.bitwise_and(iv, 3), 5)
            for j in range(16):
                q = qv[j]
                r = m * 16 + j
                rowsB[rb, r, pl.ds(0, 16)] = rowsA[ra, r, pl.ds(q, 16)]
                rowsB[rb, r, pl.ds(16, 16)] = rowsA[ra, r, pl.ds(q + 16, 16)]
            return carry
        lax.fori_loop(0, G16, ebody, 0)

    def fire_store(g, rb):
        pltpu.async_copy(
            rowsB.at[rb],
            out_hbm.at[pl.ds(pl.multiple_of(base0 + g * C, C), C)],
            osem.at[rb],
        )

    def wait_store(rb):
        pltpu.make_async_copy(
            rowsB.at[rb],
            out_hbm.at[pl.ds(pl.multiple_of(base0, C), C)],
            osem.at[rb],
        ).wait()

    for g in range(2):
        fire_idx(g, g)

    def outer(i, carry):
        for u in range(4):
            g = 4 * i + u
            ib = u
            ra = u % 2
            wait_idx(ib)
            fire_gathers(ib, ra)

            @pl.when(g + 2 < nch)
            def _():
                fire_idx(g + 2, (u + 2) % 4)

            @pl.when(g >= 1)
            def _():
                drain_gathers(1 - ra)

                @pl.when(g >= 3)
                def _():
                    wait_store(1 - ra)

                extract((u + 3) % 4, 1 - ra, 1 - ra)
                fire_store(g - 1, 1 - ra)
        return carry

    lax.fori_loop(0, nch // 4, outer, 0)
    last = (nch - 1) % 2
    drain_gathers(last)
    wait_store(last)
    extract((nch - 1) % 4, last, last)
    fire_store(nch - 1, last)
    for rb in range(2):
        wait_store(rb)


def kernel(indexes, index2vec_weight):
    n = indexes.shape[0]
    assert n % (NW * C * 4) == 0
    tblq = index2vec_weight.reshape(-1, QW)
    mesh = plsc.VectorSubcoreMesh(core_axis_name="c", subcore_axis_name="s")
    f = pl.kernel(
        _gather_body,
        out_type=jax.ShapeDtypeStruct((n, D), jnp.float32),
        mesh=mesh,
        scratch_types=[
            pltpu.VMEM((NIB, C), jnp.int32),
            pltpu.VMEM((NIB, C), jnp.int32),
            pltpu.VMEM((2, C, QW), jnp.float32),
            pltpu.VMEM((2, C, D), jnp.float32),
            pltpu.SemaphoreType.DMA((NIB,)),
            pltpu.SemaphoreType.DMA((NIB,)),
            pltpu.SemaphoreType.DMA((2,)),
            pltpu.SemaphoreType.DMA((2,)),
        ],
        compiler_params=pltpu.CompilerParams(use_tc_tiling_on_sc=True, needs_layout_passes=False),
    )
    return f(indexes, indexes >> 2, tblq)
